# speedup vs baseline: 3.0915x; 3.0915x over previous
import functools

import jax
import jax.numpy as jnp
from jax import lax
from jax.experimental import pallas as pl
from jax.experimental.pallas import tpu as pltpu
from jax.experimental.pallas import tpu_sc as plsc

N = 10000
E = 160000
D = 256
H = 128
NC = 2
NS = 16
NPAD = 10240
RPT = NPAD // NS
EPT = E // NS
CHUNK = 80
NCHUNKS = EPT // CHUNK
ZROWS = 32

_f32 = jnp.float32


def _sc_mesh():
    return plsc.VectorSubcoreMesh(
        core_axis_name="c", subcore_axis_name="s", num_cores=NC,
        num_subcores=NS)


def _deg_body(dst_hbm, deg_hbm, idx_v, ones_v, zbuf_v, acc_sh, sem):
    c = lax.axis_index("c")
    s = lax.axis_index("s")

    @pl.when(c == 0)
    def _():
        for j in range(RPT // 16):
            zbuf_v[pl.ds(j * 16, 16)] = jnp.zeros((16,), _f32)
        for j in range(CHUNK // 16):
            ones_v[pl.ds(j * 16, 16)] = jnp.ones((16,), _f32)
        pltpu.sync_copy(zbuf_v, acc_sh.at[pl.ds(s * RPT, RPT)])
        plsc.subcore_barrier()

        def body(i, carry):
            base = s * EPT + i * CHUNK
            pltpu.sync_copy(dst_hbm.at[pl.ds(base, CHUNK)], idx_v)
            pltpu.sync_copy(ones_v, acc_sh.at[idx_v], add=True)
            return carry

        lax.fori_loop(0, NCHUNKS, body, 0)
        plsc.subcore_barrier()
        pltpu.sync_copy(acc_sh.at[pl.ds(s * RPT, RPT)],
                        deg_hbm.at[pl.ds(s * RPT, RPT)])


_deg_call = pl.kernel(
    _deg_body,
    out_type=jax.ShapeDtypeStruct((NPAD,), _f32),
    mesh=_sc_mesh(),
    scratch_types=[
        pltpu.VMEM((CHUNK,), jnp.int32),
        pltpu.VMEM((CHUNK,), _f32),
        pltpu.VMEM((RPT,), _f32),
        pltpu.VMEM_SHARED((NPAD,), _f32),
        pltpu.SemaphoreType.DMA,
    ],
)


def _seg_body(y0_hbm, y1_hbm, src_hbm, dst_hbm, agg0_hbm, agg1_hbm,
              sidx_v, didx_v, rows_v, ztile_v, acc_sh, sem):
    c = lax.axis_index("c")
    s = lax.axis_index("s")

    for r in range(ZROWS):
        for j in range(H // 16):
            ztile_v[r, pl.ds(j * 16, 16)] = jnp.zeros((16,), _f32)
    for t in range(RPT // ZROWS):
        pltpu.sync_copy(ztile_v, acc_sh.at[pl.ds(s * RPT + t * ZROWS, ZROWS)])
    plsc.subcore_barrier()

    def body(i, carry):
        base = s * EPT + i * CHUNK
        pltpu.sync_copy(src_hbm.at[pl.ds(base, CHUNK)], sidx_v)
        pltpu.sync_copy(dst_hbm.at[pl.ds(base, CHUNK)], didx_v)

        @pl.when(c == 0)
        def _():
            pltpu.async_copy(y0_hbm.at[sidx_v], rows_v, sem).wait()

        @pl.when(c == 1)
        def _():
            pltpu.async_copy(y1_hbm.at[sidx_v], rows_v, sem).wait()

        pltpu.sync_copy(rows_v, acc_sh.at[didx_v], add=True)
        return carry

    lax.fori_loop(0, NCHUNKS, body, 0)
    plsc.subcore_barrier()

    rows = pl.ds(s * RPT, RPT)

    @pl.when(c == 0)
    def _():
        pltpu.sync_copy(acc_sh.at[rows], agg0_hbm.at[rows])

    @pl.when(c == 1)
    def _():
        pltpu.sync_copy(acc_sh.at[rows], agg1_hbm.at[rows])


_seg_call = pl.kernel(
    _seg_body,
    out_type=[jax.ShapeDtypeStruct((NPAD, H), _f32),
              jax.ShapeDtypeStruct((NPAD, H), _f32)],
    mesh=_sc_mesh(),
    scratch_types=[
        pltpu.VMEM((CHUNK,), jnp.int32),
        pltpu.VMEM((CHUNK,), jnp.int32),
        pltpu.VMEM((CHUNK, H), _f32),
        pltpu.VMEM((ZROWS, H), _f32),
        pltpu.VMEM_SHARED((NPAD, H), _f32),
        pltpu.SemaphoreType.DMA,
    ],
)


RB = 1000
GRID = N // RB


def _mm_first_body(x_ref, wn_ref, wr_ref, y0_ref, y1_ref, z_ref):
    h = x_ref[...]
    y = jnp.dot(h, wn_ref[...], preferred_element_type=_f32)
    y0_ref[...] = y[:, :H]
    y1_ref[...] = y[:, H:]
    z_ref[...] = jnp.dot(h, wr_ref[...], preferred_element_type=_f32)


_mm_first = pl.pallas_call(
    _mm_first_body,
    grid=(GRID,),
    in_specs=[
        pl.BlockSpec((RB, D), lambda i: (i, 0)),
        pl.BlockSpec((D, D), lambda i: (0, 0)),
        pl.BlockSpec((D, D), lambda i: (0, 0)),
    ],
    out_specs=[
        pl.BlockSpec((RB, H), lambda i: (i, 0)),
        pl.BlockSpec((RB, H), lambda i: (i, 0)),
        pl.BlockSpec((RB, D), lambda i: (i, 0)),
    ],
    out_shape=[
        jax.ShapeDtypeStruct((N, H), _f32),
        jax.ShapeDtypeStruct((N, H), _f32),
        jax.ShapeDtypeStruct((N, D), _f32),
    ],
)


def _comb_body(a0_ref, a1_ref, deg_ref, z_ref, b_ref, wn_ref, wr_ref,
               y0_ref, y1_ref, zo_ref):
    agg = jnp.concatenate([a0_ref[...], a1_ref[...]], axis=1)
    inv = 1.0 / jnp.maximum(deg_ref[...], 1.0)
    h = jnp.maximum(agg * inv + z_ref[...] + b_ref[...], 0.0)
    y = jnp.dot(h, wn_ref[...], preferred_element_type=_f32)
    y0_ref[...] = y[:, :H]
    y1_ref[...] = y[:, H:]
    zo_ref[...] = jnp.dot(h, wr_ref[...], preferred_element_type=_f32)


_comb = pl.pallas_call(
    _comb_body,
    grid=(GRID,),
    in_specs=[
        pl.BlockSpec((RB, H), lambda i: (i, 0)),
        pl.BlockSpec((RB, H), lambda i: (i, 0)),
        pl.BlockSpec((RB, 1), lambda i: (i, 0)),
        pl.BlockSpec((RB, D), lambda i: (i, 0)),
        pl.BlockSpec((1, D), lambda i: (0, 0)),
        pl.BlockSpec((D, D), lambda i: (0, 0)),
        pl.BlockSpec((D, D), lambda i: (0, 0)),
    ],
    out_specs=[
        pl.BlockSpec((RB, H), lambda i: (i, 0)),
        pl.BlockSpec((RB, H), lambda i: (i, 0)),
        pl.BlockSpec((RB, D), lambda i: (i, 0)),
    ],
    out_shape=[
        jax.ShapeDtypeStruct((N, H), _f32),
        jax.ShapeDtypeStruct((N, H), _f32),
        jax.ShapeDtypeStruct((N, D), _f32),
    ],
)

FC_PAD = 128


def _final_body(a0_ref, a1_ref, deg_ref, z_ref, b_ref, wfc_ref, bfc_ref,
                out_ref):
    agg = jnp.concatenate([a0_ref[...], a1_ref[...]], axis=1)
    inv = 1.0 / jnp.maximum(deg_ref[...], 1.0)
    h = agg * inv + z_ref[...] + b_ref[...]
    out_ref[...] = jnp.dot(h, wfc_ref[...],
                           preferred_element_type=_f32) + bfc_ref[...]


_final = pl.pallas_call(
    _final_body,
    grid=(GRID,),
    in_specs=[
        pl.BlockSpec((RB, H), lambda i: (i, 0)),
        pl.BlockSpec((RB, H), lambda i: (i, 0)),
        pl.BlockSpec((RB, 1), lambda i: (i, 0)),
        pl.BlockSpec((RB, D), lambda i: (i, 0)),
        pl.BlockSpec((1, D), lambda i: (0, 0)),
        pl.BlockSpec((D, FC_PAD), lambda i: (0, 0)),
        pl.BlockSpec((1, FC_PAD), lambda i: (0, 0)),
    ],
    out_specs=pl.BlockSpec((RB, FC_PAD), lambda i: (i, 0)),
    out_shape=jax.ShapeDtypeStruct((N, FC_PAD), _f32),
)


def kernel(x, edge_index, W_l1a_n, W_l1a_r, b_l1a, W_l1b_n, W_l1b_r, b_l1b,
           W_l2a_n, W_l2a_r, b_l2a, W_l2b_n, W_l2b_r, b_l2b, W_fc, b_fc):
    src = edge_index[0]
    dst = edge_index[1]

    deg = _deg_call(dst).reshape(NPAD, 1)[:N]

    wfc_pad = jnp.zeros((D, FC_PAD), _f32).at[:, :2].set(W_fc)
    bfc_pad = jnp.zeros((1, FC_PAD), _f32).at[0, :2].set(b_fc)

    y0, y1, z = _mm_first(x, W_l1a_n, W_l1a_r)
    a0, a1 = _seg_call(y0, y1, src, dst)
    y0, y1, z = _comb(a0[:N], a1[:N], deg, z, b_l1a.reshape(1, D),
                      W_l1b_n, W_l1b_r)
    a0, a1 = _seg_call(y0, y1, src, dst)
    y0, y1, z = _comb(a0[:N], a1[:N], deg, z, b_l1b.reshape(1, D),
                      W_l2a_n, W_l2a_r)
    a0, a1 = _seg_call(y0, y1, src, dst)
    y0, y1, z = _comb(a0[:N], a1[:N], deg, z, b_l2a.reshape(1, D),
                      W_l2b_n, W_l2b_r)
    a0, a1 = _seg_call(y0, y1, src, dst)
    out = _final(a0[:N], a1[:N], deg, z, b_l2b.reshape(1, D),
                 wfc_pad, bfc_pad)
    return out[:, :2]


# speedup vs baseline: 5.7376x; 1.8559x over previous
import jax
import jax.numpy as jnp
from jax import lax
from jax.experimental import pallas as pl
from jax.experimental.pallas import tpu as pltpu
from jax.experimental.pallas import tpu_sc as plsc

N = 10000
E = 160000
D = 256
H = 128
NC = 2
NS = 16
NPAD = 10240
RPT = NPAD // NS
EPT = E // NS
CHUNK = 80
NCHUNKS = EPT // CHUNK
NBUF = 2
NRND = NCHUNKS // NBUF
TAIL = NCHUNKS - NRND * NBUF
CHUNK_D = 80
NCHUNKS_D = EPT // CHUNK_D
WBT = 10
WBR = N // WBT

_f32 = jnp.float32


def _sc_mesh():
    return plsc.VectorSubcoreMesh(
        core_axis_name="c", subcore_axis_name="s", num_cores=NC,
        num_subcores=NS)


def _deg_body(dst_hbm, deg_hbm, idx_v, ones_v, zbuf_v, acc_sh, sem):
    c = lax.axis_index("c")
    s = lax.axis_index("s")

    @pl.when(c == 0)
    def _():
        for j in range(RPT // 16):
            zbuf_v[pl.ds(j * 16, 16)] = jnp.zeros((16,), _f32)
        for j in range(CHUNK_D // 16):
            ones_v[pl.ds(j * 16, 16)] = jnp.ones((16,), _f32)
        pltpu.sync_copy(zbuf_v, acc_sh.at[pl.ds(s * RPT, RPT)])
        plsc.subcore_barrier()

        def body(i, carry):
            base = s * EPT + i * CHUNK_D
            pltpu.sync_copy(dst_hbm.at[pl.ds(base, CHUNK_D)], idx_v)
            pltpu.sync_copy(ones_v, acc_sh.at[idx_v], add=True)
            return carry

        lax.fori_loop(0, NCHUNKS_D, body, 0)
        plsc.subcore_barrier()
        pltpu.sync_copy(acc_sh.at[pl.ds(s * RPT, RPT)],
                        deg_hbm.at[pl.ds(s * RPT, RPT)])


_deg_call = pl.kernel(
    _deg_body,
    out_type=jax.ShapeDtypeStruct((NPAD,), _f32),
    mesh=_sc_mesh(),
    scratch_types=[
        pltpu.VMEM((CHUNK_D,), jnp.int32),
        pltpu.VMEM((CHUNK_D,), _f32),
        pltpu.VMEM((RPT,), _f32),
        pltpu.VMEM_SHARED((NPAD,), _f32),
        pltpu.SemaphoreType.DMA,
    ],
)


def _seg_body(y0_hbm, y1_hbm, pidx_hbm, zrows_hbm, agg0_hbm, agg1_hbm,
              pi0, pi1, si0, si1, di0, di1, ro0, ro1, acc_sh,
              is0, is1, gs0, gs1, ss0, ss1, zsem):
    c = lax.axis_index("c")
    s = lax.axis_index("s")
    pidx = [pi0, pi1]
    sidx = [si0, si1]
    didx = [di0, di1]
    rows = [ro0, ro1]
    isem = [is0, is1]
    gsem = [gs0, gs1]
    ssem = [ss0, ss1]

    @pl.when(s < WBT)
    def _():
        pltpu.async_copy(zrows_hbm, acc_sh.at[pl.ds(s * WBR, WBR)], zsem)

    def start_pidx(i, b):
        pltpu.async_copy(pidx_hbm.at[s * NCHUNKS + i], pidx[b], isem[b])

    def wait_pidx(b):
        pltpu.make_async_copy(pidx_hbm.at[0], pidx[b], isem[b]).wait()

    def unpack(b):
        for j in range(CHUNK // 16):
            v = pidx[b][0, pl.ds(j * 16, 16)]
            sidx[b][pl.ds(j * 16, 16)] = v & 0xFFFF
            didx[b][pl.ds(j * 16, 16)] = lax.shift_right_logical(v, 16)

    def start_gather(b):
        @pl.when(c == 0)
        def _():
            pltpu.async_copy(y0_hbm.at[sidx[b]], rows[b], gsem[b])

        @pl.when(c == 1)
        def _():
            pltpu.async_copy(y1_hbm.at[sidx[b]], rows[b], gsem[b])

    def wait_gather(b):
        pltpu.make_async_copy(y0_hbm.at[sidx[b]], rows[b], gsem[b]).wait()

    def start_scatter(b):
        pltpu.async_copy(rows[b], acc_sh.at[didx[b]], ssem[b], add=True)

    def wait_scatter(b):
        pltpu.make_async_copy(rows[b], acc_sh.at[didx[b]], ssem[b]).wait()

    for b in range(NBUF):
        start_pidx(b, b)
    for b in range(NBUF):
        wait_pidx(b)
        unpack(b)
        start_pidx(NBUF + b, b)
        start_gather(b)

    @pl.when(s < WBT)
    def _():
        pltpu.make_async_copy(zrows_hbm, acc_sh.at[pl.ds(0, WBR)],
                              zsem).wait()

    plsc.subcore_barrier()

    def rnd(r, carry):
        base = r * NBUF
        for b in range(NBUF):
            wait_gather(b)
            start_scatter(b)
        for b in range(NBUF):
            nxt = base + NBUF + b

            @pl.when(nxt < NCHUNKS)
            def _():
                wait_scatter(b)
                wait_pidx(b)
                unpack(b)

                @pl.when(nxt + NBUF < NCHUNKS)
                def _():
                    start_pidx(nxt + NBUF, b)

                start_gather(b)
        return carry

    lax.fori_loop(0, NRND, rnd, 0)

    for b in range(TAIL):
        wait_gather(b)
        start_scatter(b)
    for b in range(TAIL, NBUF):
        wait_scatter(b)
    for b in range(TAIL):
        wait_scatter(b)
    plsc.subcore_barrier()

    @pl.when(s < WBT)
    def _():
        rws = pl.ds(s * WBR, WBR)

        @pl.when(c == 0)
        def _():
            pltpu.sync_copy(acc_sh.at[rws], agg0_hbm.at[rws])

        @pl.when(c == 1)
        def _():
            pltpu.sync_copy(acc_sh.at[rws], agg1_hbm.at[rws])


_seg_call = pl.kernel(
    _seg_body,
    out_type=[jax.ShapeDtypeStruct((N, H), _f32),
              jax.ShapeDtypeStruct((N, H), _f32)],
    mesh=_sc_mesh(),
    scratch_types=(
        [pltpu.VMEM((1, CHUNK), jnp.int32) for _ in range(NBUF)]
        + [pltpu.VMEM((CHUNK,), jnp.int32) for _ in range(2 * NBUF)]
        + [pltpu.VMEM((CHUNK, H), _f32) for _ in range(NBUF)]
        + [pltpu.VMEM_SHARED((N, H), _f32)]
        + [pltpu.SemaphoreType.DMA for _ in range(3 * NBUF + 1)]
    ),
)


RB = 1000
GRID = N // RB

_H_OUT = [jax.ShapeDtypeStruct((N, H), _f32),
          jax.ShapeDtypeStruct((N, H), _f32)]
_H_SPECS = [pl.BlockSpec((RB, H), lambda i: (i, 0)),
            pl.BlockSpec((RB, H), lambda i: (i, 0))]


def _comb_body(a0_ref, a1_ref, deg_ref, h0_ref, h1_ref, b_ref,
               wn_ref, wr_ref, o0_ref, o1_ref):
    inv = 1.0 / jnp.maximum(deg_ref[...], 1.0)
    mean = jnp.concatenate([a0_ref[...], a1_ref[...]], axis=1) * inv
    h = jnp.concatenate([h0_ref[...], h1_ref[...]], axis=1)
    o = (jnp.dot(mean, wn_ref[...], preferred_element_type=_f32)
         + b_ref[...]
         + jnp.dot(h, wr_ref[...], preferred_element_type=_f32))
    o = jnp.maximum(o, 0.0)
    o0_ref[...] = o[:, :H]
    o1_ref[...] = o[:, H:]


_comb = pl.pallas_call(
    _comb_body,
    grid=(GRID,),
    in_specs=_H_SPECS + _H_SPECS[:1] * 0 + [
        pl.BlockSpec((RB, 1), lambda i: (i, 0)),
    ] + _H_SPECS + [
        pl.BlockSpec((1, D), lambda i: (0, 0)),
        pl.BlockSpec((D, D), lambda i: (0, 0)),
        pl.BlockSpec((D, D), lambda i: (0, 0)),
    ],
    out_specs=_H_SPECS,
    out_shape=_H_OUT,
)

FC_PAD = 128


def _final_body(a0_ref, a1_ref, deg_ref, h0_ref, h1_ref, b_ref,
                wn_ref, wr_ref, wfc_ref, bfc_ref, out_ref):
    inv = 1.0 / jnp.maximum(deg_ref[...], 1.0)
    mean = jnp.concatenate([a0_ref[...], a1_ref[...]], axis=1) * inv
    h = jnp.concatenate([h0_ref[...], h1_ref[...]], axis=1)
    h4 = (jnp.dot(mean, wn_ref[...], preferred_element_type=_f32)
          + b_ref[...]
          + jnp.dot(h, wr_ref[...], preferred_element_type=_f32))
    out_ref[...] = jnp.dot(h4, wfc_ref[...],
                           preferred_element_type=_f32) + bfc_ref[...]


_final = pl.pallas_call(
    _final_body,
    grid=(GRID,),
    in_specs=_H_SPECS + [
        pl.BlockSpec((RB, 1), lambda i: (i, 0)),
    ] + _H_SPECS + [
        pl.BlockSpec((1, D), lambda i: (0, 0)),
        pl.BlockSpec((D, D), lambda i: (0, 0)),
        pl.BlockSpec((D, D), lambda i: (0, 0)),
        pl.BlockSpec((D, FC_PAD), lambda i: (0, 0)),
        pl.BlockSpec((1, FC_PAD), lambda i: (0, 0)),
    ],
    out_specs=pl.BlockSpec((RB, FC_PAD), lambda i: (i, 0)),
    out_shape=jax.ShapeDtypeStruct((N, FC_PAD), _f32),
)


def kernel(x, edge_index, W_l1a_n, W_l1a_r, b_l1a, W_l1b_n, W_l1b_r, b_l1b,
           W_l2a_n, W_l2a_r, b_l2a, W_l2b_n, W_l2b_r, b_l2b, W_fc, b_fc):
    src = edge_index[0]
    dst = edge_index[1]
    pidx = (src | (dst << 16)).reshape(NS * NCHUNKS, 1, CHUNK)
    zrows = jnp.zeros((WBR, H), _f32)

    deg = _deg_call(dst).reshape(NPAD, 1)[:N]

    wfc_pad = jnp.zeros((D, FC_PAD), _f32).at[:, :2].set(W_fc)
    bfc_pad = jnp.zeros((1, FC_PAD), _f32).at[0, :2].set(b_fc)

    h0, h1 = x[:, :H], x[:, H:]
    a0, a1 = _seg_call(h0, h1, pidx, zrows)
    h0, h1 = _comb(a0, a1, deg, h0, h1, b_l1a.reshape(1, D),
                   W_l1a_n, W_l1a_r)
    a0, a1 = _seg_call(h0, h1, pidx, zrows)
    h0, h1 = _comb(a0, a1, deg, h0, h1, b_l1b.reshape(1, D),
                   W_l1b_n, W_l1b_r)
    a0, a1 = _seg_call(h0, h1, pidx, zrows)
    h0, h1 = _comb(a0, a1, deg, h0, h1, b_l2a.reshape(1, D),
                   W_l2a_n, W_l2a_r)
    a0, a1 = _seg_call(h0, h1, pidx, zrows)
    out = _final(a0, a1, deg, h0, h1, b_l2b.reshape(1, D),
                 W_l2b_n, W_l2b_r, wfc_pad, bfc_pad)
    return out[:, :2]


# speedup vs baseline: 6.9888x; 1.2181x over previous
import jax
import jax.numpy as jnp
from jax import lax
from jax.experimental import pallas as pl
from jax.experimental.pallas import tpu as pltpu
from jax.experimental.pallas import tpu_sc as plsc

N = 10000
E = 160000
D = 256
H = 128
NC = 2
NS = 16
NPAD = 10240
RPT = NPAD // NS
EPT = E // NS
CHUNK = 40
NCHUNKS = EPT // CHUNK
NBUF = 4
NRND = NCHUNKS // NBUF
TAIL = NCHUNKS - NRND * NBUF
UNPACK_OFFS = list(range(0, CHUNK - 15, 16)) + (
    [CHUNK - 16] if CHUNK % 16 else [])
CHUNK_D = 80
NCHUNKS_D = EPT // CHUNK_D
WBT = 10
WBR = N // WBT

_f32 = jnp.float32


def _sc_mesh():
    return plsc.VectorSubcoreMesh(
        core_axis_name="c", subcore_axis_name="s", num_cores=NC,
        num_subcores=NS)


def _deg_body(dst_hbm, deg_hbm, idx_v, ones_v, zbuf_v, acc_sh, sem):
    c = lax.axis_index("c")
    s = lax.axis_index("s")

    @pl.when(c == 0)
    def _():
        for j in range(RPT // 16):
            zbuf_v[pl.ds(j * 16, 16)] = jnp.zeros((16,), _f32)
        for j in range(CHUNK_D // 16):
            ones_v[pl.ds(j * 16, 16)] = jnp.ones((16,), _f32)
        pltpu.sync_copy(zbuf_v, acc_sh.at[pl.ds(s * RPT, RPT)])
        plsc.subcore_barrier()

        def body(i, carry):
            base = s * EPT + i * CHUNK_D
            pltpu.sync_copy(dst_hbm.at[pl.ds(base, CHUNK_D)], idx_v)
            pltpu.sync_copy(ones_v, acc_sh.at[idx_v], add=True)
            return carry

        lax.fori_loop(0, NCHUNKS_D, body, 0)
        plsc.subcore_barrier()
        pltpu.sync_copy(acc_sh.at[pl.ds(s * RPT, RPT)],
                        deg_hbm.at[pl.ds(s * RPT, RPT)])


_deg_call = pl.kernel(
    _deg_body,
    out_type=jax.ShapeDtypeStruct((NPAD,), _f32),
    mesh=_sc_mesh(),
    scratch_types=[
        pltpu.VMEM((CHUNK_D,), jnp.int32),
        pltpu.VMEM((CHUNK_D,), _f32),
        pltpu.VMEM((RPT,), _f32),
        pltpu.VMEM_SHARED((NPAD,), _f32),
        pltpu.SemaphoreType.DMA,
    ],
)


def _seg_body(y0_hbm, y1_hbm, pidx_hbm, zrows_hbm, agg0_hbm, agg1_hbm,
              pi0, pi1, pi2, pi3, si0, si1, si2, si3, di0, di1, di2, di3,
              ro0, ro1, ro2, ro3, acc_sh,
              is0, is1, is2, is3, gs0, gs1, gs2, gs3,
              ss0, ss1, ss2, ss3, zsem):
    c = lax.axis_index("c")
    s = lax.axis_index("s")
    pidx = [pi0, pi1, pi2, pi3]
    sidx = [si0, si1, si2, si3]
    didx = [di0, di1, di2, di3]
    rows = [ro0, ro1, ro2, ro3]
    isem = [is0, is1, is2, is3]
    gsem = [gs0, gs1, gs2, gs3]
    ssem = [ss0, ss1, ss2, ss3]

    @pl.when(s < WBT)
    def _():
        pltpu.async_copy(zrows_hbm, acc_sh.at[pl.ds(s * WBR, WBR)], zsem)

    def start_pidx(i, b):
        pltpu.async_copy(pidx_hbm.at[s * NCHUNKS + i], pidx[b], isem[b])

    def wait_pidx(b):
        pltpu.make_async_copy(pidx_hbm.at[0], pidx[b], isem[b]).wait()

    def unpack(b):
        for off in UNPACK_OFFS:
            v = pidx[b][0, pl.ds(off, 16)]
            sidx[b][pl.ds(off, 16)] = v & 0xFFFF
            didx[b][pl.ds(off, 16)] = lax.shift_right_logical(v, 16)

    def start_gather(b):
        @pl.when(c == 0)
        def _():
            pltpu.async_copy(y0_hbm.at[sidx[b]], rows[b], gsem[b])

        @pl.when(c == 1)
        def _():
            pltpu.async_copy(y1_hbm.at[sidx[b]], rows[b], gsem[b])

    def wait_gather(b):
        pltpu.make_async_copy(y0_hbm.at[sidx[b]], rows[b], gsem[b]).wait()

    def start_scatter(b):
        pltpu.async_copy(rows[b], acc_sh.at[didx[b]], ssem[b], add=True)

    def wait_scatter(b):
        pltpu.make_async_copy(rows[b], acc_sh.at[didx[b]], ssem[b]).wait()

    for b in range(NBUF):
        start_pidx(b, b)
    for b in range(NBUF):
        wait_pidx(b)
        unpack(b)
        start_pidx(NBUF + b, b)
        start_gather(b)

    @pl.when(s < WBT)
    def _():
        pltpu.make_async_copy(zrows_hbm, acc_sh.at[pl.ds(0, WBR)],
                              zsem).wait()

    plsc.subcore_barrier()

    def rnd(r, carry):
        base = r * NBUF
        for b in range(NBUF):
            wait_gather(b)
            start_scatter(b)
        for b in range(NBUF):
            nxt = base + NBUF + b

            @pl.when(nxt < NCHUNKS)
            def _():
                wait_scatter(b)
                wait_pidx(b)
                unpack(b)

                @pl.when(nxt + NBUF < NCHUNKS)
                def _():
                    start_pidx(nxt + NBUF, b)

                start_gather(b)
        return carry

    lax.fori_loop(0, NRND, rnd, 0)

    for b in range(TAIL):
        wait_gather(b)
        start_scatter(b)
    for b in range(TAIL, NBUF):
        wait_scatter(b)
    for b in range(TAIL):
        wait_scatter(b)
    plsc.subcore_barrier()

    @pl.when(s < WBT)
    def _():
        rws = pl.ds(s * WBR, WBR)

        @pl.when(c == 0)
        def _():
            pltpu.sync_copy(acc_sh.at[rws], agg0_hbm.at[rws])

        @pl.when(c == 1)
        def _():
            pltpu.sync_copy(acc_sh.at[rws], agg1_hbm.at[rws])


_seg_call = pl.kernel(
    _seg_body,
    out_type=[jax.ShapeDtypeStruct((N, H), _f32),
              jax.ShapeDtypeStruct((N, H), _f32)],
    mesh=_sc_mesh(),
    scratch_types=(
        [pltpu.VMEM((1, CHUNK), jnp.int32) for _ in range(NBUF)]
        + [pltpu.VMEM((CHUNK,), jnp.int32) for _ in range(2 * NBUF)]
        + [pltpu.VMEM((CHUNK, H), _f32) for _ in range(NBUF)]
        + [pltpu.VMEM_SHARED((N, H), _f32)]
        + [pltpu.SemaphoreType.DMA for _ in range(3 * NBUF + 1)]
    ),
)


RB = 1000
GRID = N // RB

_H_OUT = [jax.ShapeDtypeStruct((N, H), _f32),
          jax.ShapeDtypeStruct((N, H), _f32)]
_H_SPECS = [pl.BlockSpec((RB, H), lambda i: (i, 0)),
            pl.BlockSpec((RB, H), lambda i: (i, 0))]


def _comb_body(a0_ref, a1_ref, deg_ref, h0_ref, h1_ref, b_ref,
               wn_ref, wr_ref, o0_ref, o1_ref):
    inv = 1.0 / jnp.maximum(deg_ref[...], 1.0)
    mean = jnp.concatenate([a0_ref[...], a1_ref[...]], axis=1) * inv
    h = jnp.concatenate([h0_ref[...], h1_ref[...]], axis=1)
    o = (jnp.dot(mean, wn_ref[...], preferred_element_type=_f32)
         + b_ref[...]
         + jnp.dot(h, wr_ref[...], preferred_element_type=_f32))
    o = jnp.maximum(o, 0.0)
    o0_ref[...] = o[:, :H]
    o1_ref[...] = o[:, H:]


_comb = pl.pallas_call(
    _comb_body,
    grid=(GRID,),
    in_specs=_H_SPECS + _H_SPECS[:1] * 0 + [
        pl.BlockSpec((RB, 1), lambda i: (i, 0)),
    ] + _H_SPECS + [
        pl.BlockSpec((1, D), lambda i: (0, 0)),
        pl.BlockSpec((D, D), lambda i: (0, 0)),
        pl.BlockSpec((D, D), lambda i: (0, 0)),
    ],
    out_specs=_H_SPECS,
    out_shape=_H_OUT,
)

FC_PAD = 128


def _final_body(a0_ref, a1_ref, deg_ref, h0_ref, h1_ref, b_ref,
                wn_ref, wr_ref, wfc_ref, bfc_ref, out_ref):
    inv = 1.0 / jnp.maximum(deg_ref[...], 1.0)
    mean = jnp.concatenate([a0_ref[...], a1_ref[...]], axis=1) * inv
    h = jnp.concatenate([h0_ref[...], h1_ref[...]], axis=1)
    h4 = (jnp.dot(mean, wn_ref[...], preferred_element_type=_f32)
          + b_ref[...]
          + jnp.dot(h, wr_ref[...], preferred_element_type=_f32))
    out_ref[...] = jnp.dot(h4, wfc_ref[...],
                           preferred_element_type=_f32) + bfc_ref[...]


_final = pl.pallas_call(
    _final_body,
    grid=(GRID,),
    in_specs=_H_SPECS + [
        pl.BlockSpec((RB, 1), lambda i: (i, 0)),
    ] + _H_SPECS + [
        pl.BlockSpec((1, D), lambda i: (0, 0)),
        pl.BlockSpec((D, D), lambda i: (0, 0)),
        pl.BlockSpec((D, D), lambda i: (0, 0)),
        pl.BlockSpec((D, FC_PAD), lambda i: (0, 0)),
        pl.BlockSpec((1, FC_PAD), lambda i: (0, 0)),
    ],
    out_specs=pl.BlockSpec((RB, FC_PAD), lambda i: (i, 0)),
    out_shape=jax.ShapeDtypeStruct((N, FC_PAD), _f32),
)


def kernel(x, edge_index, W_l1a_n, W_l1a_r, b_l1a, W_l1b_n, W_l1b_r, b_l1b,
           W_l2a_n, W_l2a_r, b_l2a, W_l2b_n, W_l2b_r, b_l2b, W_fc, b_fc):
    src = edge_index[0]
    dst = edge_index[1]
    pidx = (src | (dst << 16)).reshape(NS * NCHUNKS, 1, CHUNK)
    zrows = jnp.zeros((WBR, H), _f32)

    deg = _deg_call(dst).reshape(NPAD, 1)[:N]

    wfc_pad = jnp.zeros((D, FC_PAD), _f32).at[:, :2].set(W_fc)
    bfc_pad = jnp.zeros((1, FC_PAD), _f32).at[0, :2].set(b_fc)

    h0, h1 = x[:, :H], x[:, H:]
    a0, a1 = _seg_call(h0, h1, pidx, zrows)
    h0, h1 = _comb(a0, a1, deg, h0, h1, b_l1a.reshape(1, D),
                   W_l1a_n, W_l1a_r)
    a0, a1 = _seg_call(h0, h1, pidx, zrows)
    h0, h1 = _comb(a0, a1, deg, h0, h1, b_l1b.reshape(1, D),
                   W_l1b_n, W_l1b_r)
    a0, a1 = _seg_call(h0, h1, pidx, zrows)
    h0, h1 = _comb(a0, a1, deg, h0, h1, b_l2a.reshape(1, D),
                   W_l2a_n, W_l2a_r)
    a0, a1 = _seg_call(h0, h1, pidx, zrows)
    out = _final(a0, a1, deg, h0, h1, b_l2b.reshape(1, D),
                 W_l2b_n, W_l2b_r, wfc_pad, bfc_pad)
    return out[:, :2]


# speedup vs baseline: 7.6621x; 1.0964x over previous
import jax
import jax.numpy as jnp
from jax import lax
from jax.experimental import pallas as pl
from jax.experimental.pallas import tpu as pltpu
from jax.experimental.pallas import tpu_sc as plsc

N = 10000
E = 160000
D = 256
H = 128
NC = 2
NS = 16
NPAD = 10240
RPT = NPAD // NS
EPT = E // NS
CHUNK = 40
NCHUNKS = EPT // CHUNK
NBUF = 4
NRND = NCHUNKS // NBUF
TAIL = NCHUNKS - NRND * NBUF
UNPACK_OFFS = list(range(0, CHUNK - 15, 16)) + (
    [CHUNK - 16] if CHUNK % 16 else [])
CHUNK_D = 80
NCHUNKS_D = EPT // CHUNK_D
WBT = 10
WBR = N // WBT

_f32 = jnp.float32


def _sc_mesh():
    return plsc.VectorSubcoreMesh(
        core_axis_name="c", subcore_axis_name="s", num_cores=NC,
        num_subcores=NS)


def _make_seg_body(with_deg):
    def body(*refs):
        it = iter(refs)
        y0_hbm, y1_hbm, pidx_hbm, zrows_hbm = [next(it) for _ in range(4)]
        z1d_hbm = next(it) if with_deg else None
        agg0_hbm, agg1_hbm = next(it), next(it)
        deg_hbm = next(it) if with_deg else None
        pidx = [next(it) for _ in range(NBUF)]
        sidx = [next(it) for _ in range(NBUF)]
        didx = [next(it) for _ in range(NBUF)]
        rows = [next(it) for _ in range(NBUF)]
        ones_v = next(it) if with_deg else None
        acc_sh = next(it)
        dacc_sh = next(it) if with_deg else None
        isem = [next(it) for _ in range(NBUF)]
        gsem = [next(it) for _ in range(NBUF)]
        ssem = [next(it) for _ in range(NBUF)]
        dsem = [next(it) for _ in range(NBUF)] if with_deg else None
        zsem = next(it)

        c = lax.axis_index("c")
        s = lax.axis_index("s")

        @pl.when(s < WBT)
        def _():
            pltpu.async_copy(zrows_hbm, acc_sh.at[pl.ds(s * WBR, WBR)], zsem)

        if with_deg:
            @pl.when((c == 0) & (s == 0))
            def _():
                pltpu.async_copy(z1d_hbm, dacc_sh, zsem)
            for off in UNPACK_OFFS:
                ones_v[pl.ds(off, 16)] = jnp.ones((16,), _f32)

        def start_pidx(i, b):
            pltpu.async_copy(pidx_hbm.at[s * NCHUNKS + i], pidx[b], isem[b])

        def wait_pidx(b):
            pltpu.make_async_copy(pidx_hbm.at[0], pidx[b], isem[b]).wait()

        def unpack(b):
            for off in UNPACK_OFFS:
                v = pidx[b][0, pl.ds(off, 16)]
                sidx[b][pl.ds(off, 16)] = v & 0xFFFF
                didx[b][pl.ds(off, 16)] = lax.shift_right_logical(v, 16)

        def start_gather(b):
            @pl.when(c == 0)
            def _():
                pltpu.async_copy(y0_hbm.at[sidx[b]], rows[b], gsem[b])

            @pl.when(c == 1)
            def _():
                pltpu.async_copy(y1_hbm.at[sidx[b]], rows[b], gsem[b])

        def wait_gather(b):
            pltpu.make_async_copy(y0_hbm.at[sidx[b]], rows[b], gsem[b]).wait()

        def start_scatter(b):
            pltpu.async_copy(rows[b], acc_sh.at[didx[b]], ssem[b], add=True)
            if with_deg:
                @pl.when(c == 0)
                def _():
                    pltpu.async_copy(ones_v, dacc_sh.at[didx[b]], dsem[b],
                                     add=True)

        def wait_scatter(b):
            pltpu.make_async_copy(rows[b], acc_sh.at[didx[b]], ssem[b]).wait()
            if with_deg:
                @pl.when(c == 0)
                def _():
                    pltpu.make_async_copy(ones_v, dacc_sh.at[didx[b]],
                                          dsem[b]).wait()

        for b in range(NBUF):
            start_pidx(b, b)
        for b in range(NBUF):
            wait_pidx(b)
            unpack(b)
            start_pidx(NBUF + b, b)
            start_gather(b)

        @pl.when(s < WBT)
        def _():
            pltpu.make_async_copy(zrows_hbm, acc_sh.at[pl.ds(0, WBR)],
                                  zsem).wait()

        if with_deg:
            @pl.when((c == 0) & (s == 0))
            def _():
                pltpu.make_async_copy(z1d_hbm, dacc_sh, zsem).wait()

        plsc.subcore_barrier()

        def rnd(r, carry):
            base = r * NBUF
            for b in range(NBUF):
                wait_gather(b)
                start_scatter(b)
            for b in range(NBUF):
                nxt = base + NBUF + b

                @pl.when(nxt < NCHUNKS)
                def _():
                    wait_scatter(b)
                    wait_pidx(b)
                    unpack(b)

                    @pl.when(nxt + NBUF < NCHUNKS)
                    def _():
                        start_pidx(nxt + NBUF, b)

                    start_gather(b)
            return carry

        lax.fori_loop(0, NRND, rnd, 0)

        for b in range(TAIL):
            wait_gather(b)
            start_scatter(b)
        for b in range(TAIL, NBUF):
            wait_scatter(b)
        for b in range(TAIL):
            wait_scatter(b)
        plsc.subcore_barrier()

        @pl.when(s < WBT)
        def _():
            rws = pl.ds(s * WBR, WBR)

            @pl.when(c == 0)
            def _():
                pltpu.sync_copy(acc_sh.at[rws], agg0_hbm.at[rws])

            @pl.when(c == 1)
            def _():
                pltpu.sync_copy(acc_sh.at[rws], agg1_hbm.at[rws])

        if with_deg:
            @pl.when((c == 0) & (s == 0))
            def _():
                pltpu.sync_copy(dacc_sh, deg_hbm)

    return body


def _make_seg_call(with_deg):
    out = [jax.ShapeDtypeStruct((N, H), _f32),
           jax.ShapeDtypeStruct((N, H), _f32)]
    if with_deg:
        out = out + [jax.ShapeDtypeStruct((N,), _f32)]
    scratch = (
        [pltpu.VMEM((1, CHUNK), jnp.int32) for _ in range(NBUF)]
        + [pltpu.VMEM((CHUNK,), jnp.int32) for _ in range(2 * NBUF)]
        + [pltpu.VMEM((CHUNK, H), _f32) for _ in range(NBUF)]
        + ([pltpu.VMEM((CHUNK,), _f32)] if with_deg else [])
        + [pltpu.VMEM_SHARED((N, H), _f32)]
        + ([pltpu.VMEM_SHARED((N,), _f32)] if with_deg else [])
        + [pltpu.SemaphoreType.DMA
           for _ in range((4 if with_deg else 3) * NBUF + 1)]
    )
    return pl.kernel(
        _make_seg_body(with_deg),
        out_type=out,
        mesh=_sc_mesh(),
        scratch_types=scratch,
    )


_seg_call = _make_seg_call(False)
_segdeg_call = _make_seg_call(True)


RB = 1000
GRID = N // RB

_H_OUT = [jax.ShapeDtypeStruct((N, H), _f32),
          jax.ShapeDtypeStruct((N, H), _f32)]
_H_SPECS = [pl.BlockSpec((RB, H), lambda i: (i, 0)),
            pl.BlockSpec((RB, H), lambda i: (i, 0))]


def _comb_body(a0_ref, a1_ref, deg_ref, h0_ref, h1_ref, b_ref,
               wn_ref, wr_ref, o0_ref, o1_ref):
    inv = 1.0 / jnp.maximum(deg_ref[...], 1.0)
    mean = jnp.concatenate([a0_ref[...], a1_ref[...]], axis=1) * inv
    h = jnp.concatenate([h0_ref[...], h1_ref[...]], axis=1)
    o = (jnp.dot(mean, wn_ref[...], preferred_element_type=_f32)
         + b_ref[...]
         + jnp.dot(h, wr_ref[...], preferred_element_type=_f32))
    o = jnp.maximum(o, 0.0)
    o0_ref[...] = o[:, :H]
    o1_ref[...] = o[:, H:]


_comb = pl.pallas_call(
    _comb_body,
    grid=(GRID,),
    in_specs=_H_SPECS + _H_SPECS[:1] * 0 + [
        pl.BlockSpec((RB, 1), lambda i: (i, 0)),
    ] + _H_SPECS + [
        pl.BlockSpec((1, D), lambda i: (0, 0)),
        pl.BlockSpec((D, D), lambda i: (0, 0)),
        pl.BlockSpec((D, D), lambda i: (0, 0)),
    ],
    out_specs=_H_SPECS,
    out_shape=_H_OUT,
)

FC_PAD = 128


def _final_body(a0_ref, a1_ref, deg_ref, h0_ref, h1_ref, b_ref,
                wn_ref, wr_ref, wfc_ref, bfc_ref, out_ref):
    inv = 1.0 / jnp.maximum(deg_ref[...], 1.0)
    mean = jnp.concatenate([a0_ref[...], a1_ref[...]], axis=1) * inv
    h = jnp.concatenate([h0_ref[...], h1_ref[...]], axis=1)
    h4 = (jnp.dot(mean, wn_ref[...], preferred_element_type=_f32)
          + b_ref[...]
          + jnp.dot(h, wr_ref[...], preferred_element_type=_f32))
    out_ref[...] = jnp.dot(h4, wfc_ref[...],
                           preferred_element_type=_f32) + bfc_ref[...]


_final = pl.pallas_call(
    _final_body,
    grid=(GRID,),
    in_specs=_H_SPECS + [
        pl.BlockSpec((RB, 1), lambda i: (i, 0)),
    ] + _H_SPECS + [
        pl.BlockSpec((1, D), lambda i: (0, 0)),
        pl.BlockSpec((D, D), lambda i: (0, 0)),
        pl.BlockSpec((D, D), lambda i: (0, 0)),
        pl.BlockSpec((D, FC_PAD), lambda i: (0, 0)),
        pl.BlockSpec((1, FC_PAD), lambda i: (0, 0)),
    ],
    out_specs=pl.BlockSpec((RB, FC_PAD), lambda i: (i, 0)),
    out_shape=jax.ShapeDtypeStruct((N, FC_PAD), _f32),
)


def kernel(x, edge_index, W_l1a_n, W_l1a_r, b_l1a, W_l1b_n, W_l1b_r, b_l1b,
           W_l2a_n, W_l2a_r, b_l2a, W_l2b_n, W_l2b_r, b_l2b, W_fc, b_fc):
    src = edge_index[0]
    dst = edge_index[1]
    pidx = (src | (dst << 16)).reshape(NS * NCHUNKS, 1, CHUNK)
    zrows = jnp.zeros((WBR, H), _f32)
    z1d = jnp.zeros((N,), _f32)

    wfc_pad = jnp.zeros((D, FC_PAD), _f32).at[:, :2].set(W_fc)
    bfc_pad = jnp.zeros((1, FC_PAD), _f32).at[0, :2].set(b_fc)

    h0, h1 = x[:, :H], x[:, H:]
    a0, a1, deg = _segdeg_call(h0, h1, pidx, zrows, z1d)
    deg = deg.reshape(N, 1)
    h0, h1 = _comb(a0, a1, deg, h0, h1, b_l1a.reshape(1, D),
                   W_l1a_n, W_l1a_r)
    a0, a1 = _seg_call(h0, h1, pidx, zrows)
    h0, h1 = _comb(a0, a1, deg, h0, h1, b_l1b.reshape(1, D),
                   W_l1b_n, W_l1b_r)
    a0, a1 = _seg_call(h0, h1, pidx, zrows)
    h0, h1 = _comb(a0, a1, deg, h0, h1, b_l2a.reshape(1, D),
                   W_l2a_n, W_l2a_r)
    a0, a1 = _seg_call(h0, h1, pidx, zrows)
    out = _final(a0, a1, deg, h0, h1, b_l2b.reshape(1, D),
                 W_l2b_n, W_l2b_r, wfc_pad, bfc_pad)
    return out[:, :2]
